# baseline (speedup 1.0000x reference)
import jax
import jax.numpy as jnp
from jax import lax
from jax.experimental import pallas as pl
from jax.experimental.pallas import tpu as pltpu

B_CONV = 128
B_MLP = 512


def _shift_rows(v, shift, period, zero):
    R = v.shape[0]
    if shift == 0:
        return v
    if shift == 1:
        s = jnp.concatenate([v[1:], zero], axis=0)
        bad = period - 1
    else:
        s = jnp.concatenate([zero, v[:-1]], axis=0)
        bad = 0
    rows = lax.broadcasted_iota(jnp.int32, (R, 1), 0) % period
    return jnp.where(rows == bad, 0.0, s)


def _pool_w(pm, nw):
    return jnp.concatenate(
        [pm[:, 64 * k:64 * k + 32] for k in range(nw)], axis=1)


def _pool_h(v, sref):
    C, R, L = sref.shape
    for j in range(C):
        sref[j] = v[:, j * L:(j + 1) * L]
    e = jnp.concatenate(
        [sref[j, pl.ds(0, R // 2, stride=2), :] for j in range(C)], axis=1)
    o = jnp.concatenate(
        [sref[j, pl.ds(1, R // 2, stride=2), :] for j in range(C)], axis=1)
    return jnp.maximum(e, o)


def _conv_stage_kernel(x_ref, g1_ref, g2_ref, o_ref, s1_ref, s2_ref):
    R1 = x_ref.shape[0]
    B = R1 // 28

    X = x_ref[...]
    z1 = jnp.zeros((1, 28), jnp.float32)
    xcat = jnp.concatenate(
        [_shift_rows(X, -1, 28, z1), X, _shift_rows(X, 1, 28, z1),
         jnp.ones((R1, 1), jnp.float32)], axis=1)
    acc1 = jnp.dot(xcat.astype(jnp.bfloat16), g1_ref[...],
                   preferred_element_type=jnp.float32)

    hp = _pool_h(acc1, s1_ref)
    pm = jnp.maximum(hp[:, :864], hp[:, 32:])
    h1 = jnp.maximum(_pool_w(pm, 14), 0.0)

    R2 = R1 // 2
    z2 = jnp.zeros((1, 448), jnp.float32)
    hcat = jnp.concatenate(
        [_shift_rows(h1, -1, 14, z2), h1, _shift_rows(h1, 1, 14, z2),
         jnp.ones((R2, 1), jnp.float32)], axis=1)
    acc2 = jnp.dot(hcat.astype(jnp.bfloat16), g2_ref[...],
                   preferred_element_type=jnp.float32)

    hp2 = _pool_h(acc2, s2_ref)
    pm2 = jnp.maximum(hp2[:, :416], hp2[:, 32:])
    o_ref[...] = jnp.maximum(_pool_w(pm2, 7), 0.0)


def _conv_stage(xr, g1, g2):
    NR = xr.shape[0]
    B = B_CONV
    return pl.pallas_call(
        _conv_stage_kernel,
        out_shape=jax.ShapeDtypeStruct((NR // 4, 224), jnp.float32),
        grid_spec=pltpu.PrefetchScalarGridSpec(
            num_scalar_prefetch=0,
            grid=(NR // (B * 28),),
            in_specs=[
                pl.BlockSpec((B * 28, 28), lambda n: (n, 0)),
                pl.BlockSpec((85, 896), lambda n: (0, 0)),
                pl.BlockSpec((1345, 448), lambda n: (0, 0)),
            ],
            out_specs=pl.BlockSpec((B * 7, 224), lambda n: (n, 0)),
            scratch_shapes=[
                pltpu.VMEM((7, B * 28, 128), jnp.float32),
                pltpu.VMEM((7, B * 14, 64), jnp.float32),
            ],
        ),
        compiler_params=pltpu.CompilerParams(dimension_semantics=("parallel",)),
    )(xr, g1, g2)


def _make_g1(conv1_w, conv1_b):
    wk = conv1_w[:, 0].transpose(1, 2, 0)
    wp = jnp.arange(28)[:, None]
    w = jnp.arange(28)[None, :]
    kx = wp - w + 1
    valid = (kx >= 0) & (kx <= 2)
    kxc = jnp.clip(kx, 0, 2)
    pieces = []
    for ky in range(3):
        band = jnp.where(valid[:, :, None], wk[ky][kxc], 0.0)
        pieces.append(band.reshape(28, 896))
    pieces.append(jnp.tile(conv1_b, 28).reshape(1, 896))
    return jnp.concatenate(pieces, axis=0).astype(jnp.bfloat16)


def _make_g2(conv2_w, conv2_b):
    wk = conv2_w.transpose(2, 3, 1, 0)
    w1 = jnp.arange(14)[:, None]
    w2 = jnp.arange(14)[None, :]
    kx = w1 - w2 + 1
    valid = (kx >= 0) & (kx <= 2)
    kxc = jnp.clip(kx, 0, 2)
    pieces = []
    for ky in range(3):
        band = jnp.where(valid[:, None, :, None], wk[ky][kxc].transpose(0, 2, 1, 3), 0.0)
        pieces.append(band.reshape(448, 448))
    pieces.append(jnp.tile(conv2_b, 14).reshape(1, 448))
    return jnp.concatenate(pieces, axis=0).astype(jnp.bfloat16)


def _mlp_kernel(x_ref, w1_ref, b1_ref, w2_ref, b2_ref, o_ref):
    h = jnp.dot(x_ref[...], w1_ref[...],
                preferred_element_type=jnp.float32) + b1_ref[...]
    h = jnp.maximum(h, 0.0)
    logits = jnp.dot(h, w2_ref[...],
                     preferred_element_type=jnp.float32) + b2_ref[...]
    m = jnp.max(logits, axis=1, keepdims=True)
    s = logits - m
    lse = jnp.log(jnp.sum(jnp.exp(s), axis=1, keepdims=True))
    o_ref[...] = s - lse


def _mlp_stage(x2d, w1t, b1, w2t, b2):
    N, D = x2d.shape
    H1 = w1t.shape[1]
    C = w2t.shape[1]
    B = B_MLP
    return pl.pallas_call(
        _mlp_kernel,
        out_shape=jax.ShapeDtypeStruct((N, C), jnp.float32),
        grid_spec=pltpu.PrefetchScalarGridSpec(
            num_scalar_prefetch=0,
            grid=(N // B,),
            in_specs=[
                pl.BlockSpec((B, D), lambda n: (n, 0)),
                pl.BlockSpec((D, H1), lambda n: (0, 0)),
                pl.BlockSpec((1, H1), lambda n: (0, 0)),
                pl.BlockSpec((H1, C), lambda n: (0, 0)),
                pl.BlockSpec((1, C), lambda n: (0, 0)),
            ],
            out_specs=pl.BlockSpec((B, C), lambda n: (n, 0)),
        ),
        compiler_params=pltpu.CompilerParams(dimension_semantics=("parallel",)),
    )(x2d, w1t, b1.reshape(1, -1), w2t, b2.reshape(1, -1))


def kernel(x, conv1_w, conv1_b, conv2_w, conv2_b, fc1_w, fc1_b, fc2_w, fc2_b):
    N = x.shape[0]
    xr = x.reshape(N * 28, 28)

    g1 = _make_g1(conv1_w, conv1_b)
    g2 = _make_g2(conv2_w, conv2_b)
    h = _conv_stage(xr, g1, g2)

    hflat = h.reshape(N, 1568)
    w1t = fc1_w.reshape(500, 32, 7, 7).transpose(2, 3, 1, 0).reshape(1568, 500)
    w2t = fc2_w.T
    return _mlp_stage(hflat, w1t, fc1_b, w2t, fc2_b)


# speedup vs baseline: 1.3958x; 1.3958x over previous
import jax
import jax.numpy as jnp
from jax import lax
from jax.experimental import pallas as pl
from jax.experimental.pallas import tpu as pltpu

B_CONV = 128
B_MLP = 512


def _shift_rows(v, shift, period, zero):
    R = v.shape[0]
    if shift == 0:
        return v
    if shift == 1:
        s = jnp.concatenate([v[1:], zero], axis=0)
        bad = period - 1
    else:
        s = jnp.concatenate([zero, v[:-1]], axis=0)
        bad = 0
    rows = lax.broadcasted_iota(jnp.int32, (R, 1), 0) % period
    return jnp.where(rows == bad, 0.0, s)


def _pool_w(pm, nw):
    return jnp.concatenate(
        [pm[:, 64 * k:64 * k + 32] for k in range(nw)], axis=1)


def _pool_h(v, sref):
    C, R, L = sref.shape
    for j in range(C):
        sref[j] = v[:, j * L:(j + 1) * L]
    e = jnp.concatenate(
        [sref[j, pl.ds(0, R // 2, stride=2), :] for j in range(C)], axis=1)
    o = jnp.concatenate(
        [sref[j, pl.ds(1, R // 2, stride=2), :] for j in range(C)], axis=1)
    return jnp.maximum(e, o)


def _conv_stage_kernel(x_ref, g1_ref, g2_ref, o_ref, s1_ref, s2_ref):
    R1 = x_ref.shape[0]
    B = R1 // 28

    X = x_ref[...]
    z1 = jnp.zeros((1, 28), jnp.float32)
    xcat = jnp.concatenate(
        [_shift_rows(X, -1, 28, z1), X, _shift_rows(X, 1, 28, z1),
         jnp.ones((R1, 1), jnp.float32)], axis=1)
    acc1 = jnp.dot(xcat, g1_ref[...],
                   preferred_element_type=jnp.float32)

    hp = _pool_h(acc1, s1_ref)
    pm = jnp.maximum(hp[:, :864], hp[:, 32:])
    h1 = jnp.maximum(_pool_w(pm, 14), 0.0)

    R2 = R1 // 2
    z2 = jnp.zeros((1, 448), jnp.float32)
    hcat = jnp.concatenate(
        [_shift_rows(h1, -1, 14, z2), h1, _shift_rows(h1, 1, 14, z2),
         jnp.ones((R2, 1), jnp.float32)], axis=1)
    acc2 = jnp.dot(hcat, g2_ref[...],
                   preferred_element_type=jnp.float32)

    hp2 = _pool_h(acc2, s2_ref)
    pm2 = jnp.maximum(hp2[:, :416], hp2[:, 32:])
    o_ref[...] = jnp.maximum(_pool_w(pm2, 7), 0.0)


def _conv_stage(xr, g1, g2):
    NR = xr.shape[0]
    B = B_CONV
    return pl.pallas_call(
        _conv_stage_kernel,
        out_shape=jax.ShapeDtypeStruct((NR // 4, 224), jnp.float32),
        grid_spec=pltpu.PrefetchScalarGridSpec(
            num_scalar_prefetch=0,
            grid=(NR // (B * 28),),
            in_specs=[
                pl.BlockSpec((B * 28, 28), lambda n: (n, 0)),
                pl.BlockSpec((85, 896), lambda n: (0, 0)),
                pl.BlockSpec((1345, 448), lambda n: (0, 0)),
            ],
            out_specs=pl.BlockSpec((B * 7, 224), lambda n: (n, 0)),
            scratch_shapes=[
                pltpu.VMEM((7, B * 28, 128), jnp.float32),
                pltpu.VMEM((7, B * 14, 64), jnp.float32),
            ],
        ),
        compiler_params=pltpu.CompilerParams(dimension_semantics=("parallel",)),
    )(xr, g1, g2)


def _make_g1(conv1_w, conv1_b):
    wk = conv1_w[:, 0].transpose(1, 2, 0)
    wp = jnp.arange(28)[:, None]
    w = jnp.arange(28)[None, :]
    kx = wp - w + 1
    valid = (kx >= 0) & (kx <= 2)
    kxc = jnp.clip(kx, 0, 2)
    pieces = []
    for ky in range(3):
        band = jnp.where(valid[:, :, None], wk[ky][kxc], 0.0)
        pieces.append(band.reshape(28, 896))
    pieces.append(jnp.tile(conv1_b, 28).reshape(1, 896))
    return jnp.concatenate(pieces, axis=0)


def _make_g2(conv2_w, conv2_b):
    wk = conv2_w.transpose(2, 3, 1, 0)
    w1 = jnp.arange(14)[:, None]
    w2 = jnp.arange(14)[None, :]
    kx = w1 - w2 + 1
    valid = (kx >= 0) & (kx <= 2)
    kxc = jnp.clip(kx, 0, 2)
    pieces = []
    for ky in range(3):
        band = jnp.where(valid[:, None, :, None], wk[ky][kxc].transpose(0, 2, 1, 3), 0.0)
        pieces.append(band.reshape(448, 448))
    pieces.append(jnp.tile(conv2_b, 14).reshape(1, 448))
    return jnp.concatenate(pieces, axis=0)


def _mlp_kernel(x_ref, w1_ref, b1_ref, w2_ref, b2_ref, o_ref, s_ref):
    R = x_ref.shape[0]
    B = R // 7
    s_ref[0] = x_ref[:, :112]
    s_ref[1] = x_ref[:, 112:]
    pieces = []
    for hh in range(7):
        pieces.append(s_ref[0, pl.ds(hh, B, stride=7), :])
        pieces.append(s_ref[1, pl.ds(hh, B, stride=7), :])
    xb = jnp.concatenate(pieces, axis=1)
    h = jnp.dot(xb, w1_ref[...],
                preferred_element_type=jnp.float32) + b1_ref[...]
    h = jnp.maximum(h, 0.0)
    logits = jnp.dot(h, w2_ref[...],
                     preferred_element_type=jnp.float32) + b2_ref[...]
    m = jnp.max(logits, axis=1, keepdims=True)
    s = logits - m
    lse = jnp.log(jnp.sum(jnp.exp(s), axis=1, keepdims=True))
    o_ref[...] = s - lse


def _mlp_stage(xrows, w1t, b1, w2t, b2):
    NR, L = xrows.shape
    N = NR // 7
    H1 = w1t.shape[1]
    C = w2t.shape[1]
    B = B_MLP
    return pl.pallas_call(
        _mlp_kernel,
        out_shape=jax.ShapeDtypeStruct((N, C), jnp.float32),
        grid_spec=pltpu.PrefetchScalarGridSpec(
            num_scalar_prefetch=0,
            grid=(N // B,),
            in_specs=[
                pl.BlockSpec((B * 7, L), lambda n: (n, 0)),
                pl.BlockSpec((7 * L, H1), lambda n: (0, 0)),
                pl.BlockSpec((1, H1), lambda n: (0, 0)),
                pl.BlockSpec((H1, C), lambda n: (0, 0)),
                pl.BlockSpec((1, C), lambda n: (0, 0)),
            ],
            out_specs=pl.BlockSpec((B, C), lambda n: (n, 0)),
            scratch_shapes=[
                pltpu.VMEM((2, B * 7, 112), jnp.float32),
            ],
        ),
        compiler_params=pltpu.CompilerParams(dimension_semantics=("parallel",)),
    )(xrows, w1t, b1.reshape(1, -1), w2t, b2.reshape(1, -1))


def kernel(x, conv1_w, conv1_b, conv2_w, conv2_b, fc1_w, fc1_b, fc2_w, fc2_b):
    N = x.shape[0]
    xr = x.reshape(N * 28, 28)

    g1 = _make_g1(conv1_w, conv1_b)
    g2 = _make_g2(conv2_w, conv2_b)
    h = _conv_stage(xr, g1, g2)

    w1t = fc1_w.reshape(500, 32, 7, 7).transpose(2, 3, 1, 0).reshape(1568, 500)
    w2t = fc2_w.T
    return _mlp_stage(h, w1t, fc1_b, w2t, fc2_b)


# speedup vs baseline: 1.5623x; 1.1193x over previous
import jax
import jax.numpy as jnp
from jax import lax
from jax.experimental import pallas as pl
from jax.experimental.pallas import tpu as pltpu

B_CONV = 128
B_MLP = 512


def _shift_rows(v, shift, period, zero):
    R = v.shape[0]
    if shift == 0:
        return v
    if shift == 1:
        s = jnp.concatenate([v[1:], zero], axis=0)
        bad = period - 1
    else:
        s = jnp.concatenate([zero, v[:-1]], axis=0)
        bad = 0
    rows = lax.broadcasted_iota(jnp.int32, (R, 1), 0) % period
    return jnp.where(rows == bad, 0.0, s)


def _pool_w(pm, nw):
    return jnp.concatenate(
        [pm[:, 64 * k:64 * k + 32] for k in range(nw)], axis=1)


def _pool_h(v, sref):
    C, R, L = sref.shape
    for j in range(C):
        sref[j] = v[:, j * L:(j + 1) * L]
    e = jnp.concatenate(
        [sref[j, pl.ds(0, R // 2, stride=2), :] for j in range(C)], axis=1)
    o = jnp.concatenate(
        [sref[j, pl.ds(1, R // 2, stride=2), :] for j in range(C)], axis=1)
    return jnp.maximum(e, o)


def _conv_stage_kernel(x_ref, g1_ref, g2_ref, o_ref, s1_ref, s2_ref):
    B = x_ref.shape[0]
    R1 = B * 28

    X = x_ref[...].reshape(R1, 28)
    z1 = jnp.zeros((1, 28), jnp.float32)
    xcat = jnp.concatenate(
        [_shift_rows(X, -1, 28, z1), X, _shift_rows(X, 1, 28, z1),
         jnp.ones((R1, 1), jnp.float32)], axis=1)
    acc1 = jnp.dot(xcat, g1_ref[...],
                   preferred_element_type=jnp.float32)

    hp = _pool_h(acc1, s1_ref)
    pm = jnp.maximum(hp[:, :864], hp[:, 32:])
    h1 = jnp.maximum(_pool_w(pm, 14), 0.0)

    R2 = R1 // 2
    z2 = jnp.zeros((1, 448), jnp.float32)
    hcat = jnp.concatenate(
        [_shift_rows(h1, -1, 14, z2), h1, _shift_rows(h1, 1, 14, z2),
         jnp.ones((R2, 1), jnp.float32)], axis=1)
    acc2 = jnp.dot(hcat, g2_ref[...],
                   preferred_element_type=jnp.float32)

    hp2 = _pool_h(acc2, s2_ref)
    pm2 = jnp.maximum(hp2[:, :416], hp2[:, 32:])
    o_ref[...] = jnp.maximum(_pool_w(pm2, 7), 0.0)


def _conv_stage(xr, g1, g2):
    NR = xr.shape[0] * 28
    B = B_CONV
    return pl.pallas_call(
        _conv_stage_kernel,
        out_shape=jax.ShapeDtypeStruct((NR // 4, 224), jnp.float32),
        grid_spec=pltpu.PrefetchScalarGridSpec(
            num_scalar_prefetch=0,
            grid=(NR // (B * 28),),
            in_specs=[
                pl.BlockSpec((B, 28, 28), lambda n: (n, 0, 0)),
                pl.BlockSpec((85, 896), lambda n: (0, 0)),
                pl.BlockSpec((1345, 448), lambda n: (0, 0)),
            ],
            out_specs=pl.BlockSpec((B * 7, 224), lambda n: (n, 0)),
            scratch_shapes=[
                pltpu.VMEM((7, B * 28, 128), jnp.float32),
                pltpu.VMEM((7, B * 14, 64), jnp.float32),
            ],
        ),
        compiler_params=pltpu.CompilerParams(dimension_semantics=("parallel",)),
    )(xr, g1, g2)


def _make_g1(conv1_w, conv1_b):
    wk = conv1_w[:, 0].transpose(1, 2, 0)
    wp = jnp.arange(28)[:, None]
    w = jnp.arange(28)[None, :]
    kx = wp - w + 1
    valid = (kx >= 0) & (kx <= 2)
    kxc = jnp.clip(kx, 0, 2)
    pieces = []
    for ky in range(3):
        band = jnp.where(valid[:, :, None], wk[ky][kxc], 0.0)
        pieces.append(band.reshape(28, 896))
    pieces.append(jnp.tile(conv1_b, 28).reshape(1, 896))
    return jnp.concatenate(pieces, axis=0)


def _make_g2(conv2_w, conv2_b):
    wk = conv2_w.transpose(2, 3, 1, 0)
    w1 = jnp.arange(14)[:, None]
    w2 = jnp.arange(14)[None, :]
    kx = w1 - w2 + 1
    valid = (kx >= 0) & (kx <= 2)
    kxc = jnp.clip(kx, 0, 2)
    pieces = []
    for ky in range(3):
        band = jnp.where(valid[:, None, :, None], wk[ky][kxc].transpose(0, 2, 1, 3), 0.0)
        pieces.append(band.reshape(448, 448))
    pieces.append(jnp.tile(conv2_b, 14).reshape(1, 448))
    return jnp.concatenate(pieces, axis=0)


def _mlp_kernel(x_ref, w1_ref, b1_ref, w2_ref, b2_ref, o_ref, s_ref):
    R = x_ref.shape[0]
    B = R // 7
    s_ref[0] = x_ref[:, :112]
    s_ref[1] = x_ref[:, 112:]
    pieces = []
    for hh in range(7):
        pieces.append(s_ref[0, pl.ds(hh, B, stride=7), :])
        pieces.append(s_ref[1, pl.ds(hh, B, stride=7), :])
    xb = jnp.concatenate(pieces, axis=1)
    h = jnp.dot(xb, w1_ref[...],
                preferred_element_type=jnp.float32) + b1_ref[...]
    h = jnp.maximum(h, 0.0)
    logits = jnp.dot(h, w2_ref[...],
                     preferred_element_type=jnp.float32) + b2_ref[...]
    m = jnp.max(logits, axis=1, keepdims=True)
    s = logits - m
    lse = jnp.log(jnp.sum(jnp.exp(s), axis=1, keepdims=True))
    o_ref[...] = s - lse


def _mlp_stage(xrows, w1t, b1, w2t, b2):
    NR, L = xrows.shape
    N = NR // 7
    H1 = w1t.shape[1]
    C = w2t.shape[1]
    B = B_MLP
    return pl.pallas_call(
        _mlp_kernel,
        out_shape=jax.ShapeDtypeStruct((N, C), jnp.float32),
        grid_spec=pltpu.PrefetchScalarGridSpec(
            num_scalar_prefetch=0,
            grid=(N // B,),
            in_specs=[
                pl.BlockSpec((B * 7, L), lambda n: (n, 0)),
                pl.BlockSpec((7 * L, H1), lambda n: (0, 0)),
                pl.BlockSpec((1, H1), lambda n: (0, 0)),
                pl.BlockSpec((H1, C), lambda n: (0, 0)),
                pl.BlockSpec((1, C), lambda n: (0, 0)),
            ],
            out_specs=pl.BlockSpec((B, C), lambda n: (n, 0)),
            scratch_shapes=[
                pltpu.VMEM((2, B * 7, 112), jnp.float32),
            ],
        ),
        compiler_params=pltpu.CompilerParams(dimension_semantics=("parallel",)),
    )(xrows, w1t, b1.reshape(1, -1), w2t, b2.reshape(1, -1))


def kernel(x, conv1_w, conv1_b, conv2_w, conv2_b, fc1_w, fc1_b, fc2_w, fc2_b):
    N = x.shape[0]
    xr = x.reshape(N, 28, 28)

    g1 = _make_g1(conv1_w, conv1_b)
    g2 = _make_g2(conv2_w, conv2_b)
    h = _conv_stage(xr, g1, g2)

    w1t = fc1_w.reshape(500, 32, 7, 7).transpose(2, 3, 1, 0).reshape(1568, 500)
    w2t = fc2_w.T
    return _mlp_stage(h, w1t, fc1_b, w2t, fc2_b)


# speedup vs baseline: 1.5666x; 1.0027x over previous
import jax
import jax.numpy as jnp
from jax import lax
from jax.experimental import pallas as pl
from jax.experimental.pallas import tpu as pltpu

B_CONV = 128
B_MLP = 1024


def _shift_rows(v, shift, period, zero):
    R = v.shape[0]
    if shift == 0:
        return v
    if shift == 1:
        s = jnp.concatenate([v[1:], zero], axis=0)
        bad = period - 1
    else:
        s = jnp.concatenate([zero, v[:-1]], axis=0)
        bad = 0
    rows = lax.broadcasted_iota(jnp.int32, (R, 1), 0) % period
    return jnp.where(rows == bad, 0.0, s)


def _pool_w(pm, nw):
    return jnp.concatenate(
        [pm[:, 64 * k:64 * k + 32] for k in range(nw)], axis=1)


def _pool_h(v, sref):
    C, R, L = sref.shape
    for j in range(C):
        sref[j] = v[:, j * L:(j + 1) * L]
    e = jnp.concatenate(
        [sref[j, pl.ds(0, R // 2, stride=2), :] for j in range(C)], axis=1)
    o = jnp.concatenate(
        [sref[j, pl.ds(1, R // 2, stride=2), :] for j in range(C)], axis=1)
    return jnp.maximum(e, o)


def _conv_stage_kernel(x_ref, g1_ref, g2_ref, o_ref, s1_ref, s2_ref):
    B = x_ref.shape[0]
    R1 = B * 28

    X = x_ref[...].reshape(R1, 28)
    z1 = jnp.zeros((1, 28), jnp.float32)
    xcat = jnp.concatenate(
        [_shift_rows(X, -1, 28, z1), X, _shift_rows(X, 1, 28, z1),
         jnp.ones((R1, 1), jnp.float32)], axis=1)
    acc1 = jnp.dot(xcat, g1_ref[...],
                   preferred_element_type=jnp.float32)

    hp = _pool_h(acc1, s1_ref)
    pm = jnp.maximum(hp[:, :864], hp[:, 32:])
    h1 = jnp.maximum(_pool_w(pm, 14), 0.0)

    R2 = R1 // 2
    z2 = jnp.zeros((1, 448), jnp.float32)
    hcat = jnp.concatenate(
        [_shift_rows(h1, -1, 14, z2), h1, _shift_rows(h1, 1, 14, z2),
         jnp.ones((R2, 1), jnp.float32)], axis=1)
    acc2 = jnp.dot(hcat, g2_ref[...],
                   preferred_element_type=jnp.float32)

    hp2 = _pool_h(acc2, s2_ref)
    pm2 = jnp.maximum(hp2[:, :416], hp2[:, 32:])
    o_ref[...] = jnp.maximum(_pool_w(pm2, 7), 0.0)


def _conv_stage(xr, g1, g2):
    NR = xr.shape[0] * 28
    B = B_CONV
    return pl.pallas_call(
        _conv_stage_kernel,
        out_shape=jax.ShapeDtypeStruct((NR // 4, 224), jnp.float32),
        grid_spec=pltpu.PrefetchScalarGridSpec(
            num_scalar_prefetch=0,
            grid=(NR // (B * 28),),
            in_specs=[
                pl.BlockSpec((B, 28, 28), lambda n: (n, 0, 0)),
                pl.BlockSpec((85, 896), lambda n: (0, 0)),
                pl.BlockSpec((1345, 448), lambda n: (0, 0)),
            ],
            out_specs=pl.BlockSpec((B * 7, 224), lambda n: (n, 0)),
            scratch_shapes=[
                pltpu.VMEM((7, B * 28, 128), jnp.float32),
                pltpu.VMEM((7, B * 14, 64), jnp.float32),
            ],
        ),
        compiler_params=pltpu.CompilerParams(dimension_semantics=("parallel",)),
    )(xr, g1, g2)


def _make_g1(conv1_w, conv1_b):
    wk = conv1_w[:, 0].transpose(1, 2, 0)
    wp = jnp.arange(28)[:, None]
    w = jnp.arange(28)[None, :]
    kx = wp - w + 1
    valid = (kx >= 0) & (kx <= 2)
    kxc = jnp.clip(kx, 0, 2)
    pieces = []
    for ky in range(3):
        band = jnp.where(valid[:, :, None], wk[ky][kxc], 0.0)
        pieces.append(band.reshape(28, 896))
    pieces.append(jnp.tile(conv1_b, 28).reshape(1, 896))
    return jnp.concatenate(pieces, axis=0)


def _make_g2(conv2_w, conv2_b):
    wk = conv2_w.transpose(2, 3, 1, 0)
    w1 = jnp.arange(14)[:, None]
    w2 = jnp.arange(14)[None, :]
    kx = w1 - w2 + 1
    valid = (kx >= 0) & (kx <= 2)
    kxc = jnp.clip(kx, 0, 2)
    pieces = []
    for ky in range(3):
        band = jnp.where(valid[:, None, :, None], wk[ky][kxc].transpose(0, 2, 1, 3), 0.0)
        pieces.append(band.reshape(448, 448))
    pieces.append(jnp.tile(conv2_b, 14).reshape(1, 448))
    return jnp.concatenate(pieces, axis=0)


def _mlp_kernel(x_ref, w1_ref, b1_ref, w2_ref, b2_ref, o_ref, s_ref):
    R = x_ref.shape[0]
    B = R // 7
    s_ref[0] = x_ref[:, :112]
    s_ref[1] = x_ref[:, 112:]
    pieces = []
    for hh in range(7):
        pieces.append(s_ref[0, pl.ds(hh, B, stride=7), :])
        pieces.append(s_ref[1, pl.ds(hh, B, stride=7), :])
    xb = jnp.concatenate(pieces, axis=1)
    h = jnp.dot(xb, w1_ref[...],
                preferred_element_type=jnp.float32) + b1_ref[...]
    h = jnp.maximum(h, 0.0)
    logits = jnp.dot(h, w2_ref[...],
                     preferred_element_type=jnp.float32) + b2_ref[...]
    m = jnp.max(logits, axis=1, keepdims=True)
    s = logits - m
    lse = jnp.log(jnp.sum(jnp.exp(s), axis=1, keepdims=True))
    o_ref[...] = s - lse


def _mlp_stage(xrows, w1t, b1, w2t, b2):
    NR, L = xrows.shape
    N = NR // 7
    H1 = w1t.shape[1]
    C = w2t.shape[1]
    B = B_MLP
    return pl.pallas_call(
        _mlp_kernel,
        out_shape=jax.ShapeDtypeStruct((N, C), jnp.float32),
        grid_spec=pltpu.PrefetchScalarGridSpec(
            num_scalar_prefetch=0,
            grid=(N // B,),
            in_specs=[
                pl.BlockSpec((B * 7, L), lambda n: (n, 0)),
                pl.BlockSpec((7 * L, H1), lambda n: (0, 0)),
                pl.BlockSpec((1, H1), lambda n: (0, 0)),
                pl.BlockSpec((H1, C), lambda n: (0, 0)),
                pl.BlockSpec((1, C), lambda n: (0, 0)),
            ],
            out_specs=pl.BlockSpec((B, C), lambda n: (n, 0)),
            scratch_shapes=[
                pltpu.VMEM((2, B * 7, 112), jnp.float32),
            ],
        ),
        compiler_params=pltpu.CompilerParams(dimension_semantics=("parallel",)),
    )(xrows, w1t, b1.reshape(1, -1), w2t, b2.reshape(1, -1))


def kernel(x, conv1_w, conv1_b, conv2_w, conv2_b, fc1_w, fc1_b, fc2_w, fc2_b):
    N = x.shape[0]
    xr = x.reshape(N, 28, 28)

    g1 = _make_g1(conv1_w, conv1_b)
    g2 = _make_g2(conv2_w, conv2_b)
    h = _conv_stage(xr, g1, g2)

    w1t = fc1_w.reshape(500, 32, 7, 7).transpose(2, 3, 1, 0).reshape(1568, 500)
    w2t = fc2_w.T
    return _mlp_stage(h, w1t, fc1_b, w2t, fc2_b)
